# baseline (speedup 1.0000x reference)
import functools

import jax
import jax.numpy as jnp
from jax import lax
from jax.experimental import pallas as pl
from jax.experimental.pallas import tpu as pltpu
from jax.experimental.pallas import tpu_sc as plsc

DIM = 64
LANES = 16
NBUF = 2


def _make_sc_kernel(batch: int, pos_len: int):
    nb_blocks = 16
    s_halves = 2
    blk = batch // nb_blocks
    s_per_half = pos_len // s_halves
    mesh = plsc.VectorSubcoreMesh(core_axis_name="c", subcore_axis_name="s")

    @functools.partial(
        pl.kernel,
        mesh=mesh,
        compiler_params=pltpu.CompilerParams(
            use_tc_tiling_on_sc=False, needs_layout_passes=False),
        out_type=jax.ShapeDtypeStruct((pos_len, DIM, batch), jnp.float32),
        scratch_types=[
            [pltpu.VMEM((blk,), jnp.int32)] * NBUF,
            [pltpu.VMEM((blk,), jnp.int32)] * NBUF,
            [pltpu.VMEM((blk,), jnp.int32)] * NBUF,
            [pltpu.VMEM((blk, DIM), jnp.float32)] * NBUF,
            [pltpu.VMEM((DIM, blk), jnp.float32)] * NBUF,
            pltpu.VMEM((pos_len, DIM), jnp.float32),
            [pltpu.SemaphoreType.DMA] * NBUF,
            [pltpu.SemaphoreType.DMA] * NBUF,
        ],
    )
    def sc_kernel(idx_a_hbm, idx_b_hbm, idx_c_hbm, tab_a_hbm, tab_b_hbm,
                  tab_c_hbm, pos_hbm, out_hbm,
                  idx_a, idx_b, idx_c, acc, acc_t, pos_v, sem_g, sem_o):
        wid = lax.axis_index("s") * 2 + lax.axis_index("c")
        s_half = wid // nb_blocks
        nb0 = (wid % nb_blocks) * blk
        s0 = s_half * s_per_half

        pltpu.sync_copy(pos_hbm, pos_v)

        iota = lax.iota(jnp.int32, LANES)
        gathers = [None] * NBUF
        out_cp = [None] * NBUF

        def stage(t, p):
            s = s0 + t
            base = s * batch + nb0
            pltpu.sync_copy(idx_a_hbm.at[pl.ds(base, blk)], idx_a[p])
            pltpu.sync_copy(idx_b_hbm.at[pl.ds(base, blk)], idx_b[p])
            pltpu.sync_copy(idx_c_hbm.at[pl.ds(base, blk)], idx_c[p])

            def init_row(r, carry):
                for k in range(DIM // LANES):
                    sl = pl.ds(k * LANES, LANES)
                    acc[p][r, sl] = pos_v[s, sl]
                return carry

            lax.fori_loop(0, blk, init_row, 0)
            gathers[p] = [
                pltpu.async_copy(tab_a_hbm.at[idx_a[p]], acc[p], sem_g[p], add=True),
                pltpu.async_copy(tab_b_hbm.at[idx_b[p]], acc[p], sem_g[p], add=True),
                pltpu.async_copy(tab_c_hbm.at[idx_c[p]], acc[p], sem_g[p], add=True),
            ]

        def finish(t, q):
            s = s0 + t
            for d in gathers[q]:
                d.wait()

            def tr_col(c, carry):
                cs = jnp.full((LANES,), c, jnp.int32)
                for k in range(blk // LANES):
                    rows = iota + (k * LANES)
                    v = plsc.load_gather(acc[q], [rows, cs])
                    acc_t[q][c, pl.ds(k * LANES, LANES)] = v
                return carry

            lax.fori_loop(0, DIM, tr_col, 0)
            out_cp[q] = pltpu.async_copy(
                acc_t[q], out_hbm.at[s, :, pl.ds(nb0, blk)], sem_o[q])

        for t in range(s_per_half):
            p = t % NBUF
            if out_cp[p] is not None:
                out_cp[p].wait()
            stage(t, p)
            if t >= 1:
                finish(t - 1, (t - 1) % NBUF)
        last = s_per_half - 1
        finish(last, last % NBUF)
        out_cp[last % NBUF].wait()
        out_cp[(last - 1) % NBUF].wait()

    return sc_kernel


def kernel(feat_item, feat_category, feat_brand, positions,
           table_item, table_category, table_brand, table_position):
    batch, pos_len = feat_item.shape

    idx_a = feat_item.T.reshape(-1)
    idx_b = feat_category.T.reshape(-1)
    idx_c = feat_brand.T.reshape(-1)
    pos_rows = jnp.take(table_position, positions, axis=0)

    sc = _make_sc_kernel(batch, pos_len)
    out_t = sc(idx_a, idx_b, idx_c, table_item, table_category,
               table_brand, pos_rows)
    return jnp.transpose(out_t, (2, 0, 1))


# speedup vs baseline: 1.0657x; 1.0657x over previous
import functools

import jax
import jax.numpy as jnp
from jax import lax
from jax.experimental import pallas as pl
from jax.experimental.pallas import tpu as pltpu
from jax.experimental.pallas import tpu_sc as plsc

DIM = 64
LANES = 16
NBUF = 2


def _make_sc_kernel(batch: int, pos_len: int):
    nb_blocks = 16
    s_halves = 2
    blk = batch // nb_blocks
    s_per_half = pos_len // s_halves
    mesh = plsc.VectorSubcoreMesh(core_axis_name="c", subcore_axis_name="s")

    @functools.partial(
        pl.kernel,
        mesh=mesh,
        compiler_params=pltpu.CompilerParams(
            use_tc_tiling_on_sc=False, needs_layout_passes=False),
        out_type=jax.ShapeDtypeStruct((pos_len, DIM, batch), jnp.float32),
        scratch_types=[
            [pltpu.VMEM((blk,), jnp.int32)] * NBUF,
            [pltpu.VMEM((blk,), jnp.int32)] * NBUF,
            [pltpu.VMEM((blk,), jnp.int32)] * NBUF,
            [pltpu.VMEM((blk, DIM), jnp.float32)] * NBUF,
            [pltpu.VMEM((DIM, blk), jnp.float32)] * NBUF,
            [pltpu.SemaphoreType.DMA] * NBUF,
            [pltpu.SemaphoreType.DMA] * NBUF,
        ],
    )
    def sc_kernel(idx_a_hbm, idx_b_hbm, idx_c_hbm, tab_a_hbm, tab_b_hbm,
                  tab_c_hbm, pos_hbm, out_hbm,
                  idx_a, idx_b, idx_c, acc, acc_t, sem_g, sem_o):
        wid = lax.axis_index("s") * 2 + lax.axis_index("c")
        s_half = wid // nb_blocks
        nb0 = (wid % nb_blocks) * blk
        s0 = s_half * s_per_half

        iota = lax.iota(jnp.int32, LANES)
        gathers = [None] * NBUF
        out_cp = [None] * NBUF

        def stage(t, p):
            s = s0 + t
            base = s * batch + nb0
            pltpu.sync_copy(idx_a_hbm.at[pl.ds(base, blk)], idx_a[p])
            pltpu.sync_copy(idx_b_hbm.at[pl.ds(base, blk)], idx_b[p])
            pltpu.sync_copy(idx_c_hbm.at[pl.ds(base, blk)], idx_c[p])

            pltpu.sync_copy(pos_hbm.at[s], acc[p])
            gathers[p] = [
                pltpu.async_copy(tab_a_hbm.at[idx_a[p]], acc[p], sem_g[p], add=True),
                pltpu.async_copy(tab_b_hbm.at[idx_b[p]], acc[p], sem_g[p], add=True),
                pltpu.async_copy(tab_c_hbm.at[idx_c[p]], acc[p], sem_g[p], add=True),
            ]

        rows_k = [iota + (k * LANES) for k in range(blk // LANES)]

        def finish(t, q):
            s = s0 + t
            for d in gathers[q]:
                d.wait()

            def tr_col(c, carry):
                cs = jnp.full((LANES,), c, jnp.int32)
                for k in range(blk // LANES):
                    v = plsc.load_gather(acc[q], [rows_k[k], cs])
                    acc_t[q][c, pl.ds(k * LANES, LANES)] = v
                return carry

            lax.fori_loop(0, DIM, tr_col, 0)
            out_cp[q] = pltpu.async_copy(
                acc_t[q], out_hbm.at[s, :, pl.ds(nb0, blk)], sem_o[q])

        for t in range(s_per_half):
            p = t % NBUF
            if out_cp[p] is not None:
                out_cp[p].wait()
            stage(t, p)
            if t >= 1:
                finish(t - 1, (t - 1) % NBUF)
        last = s_per_half - 1
        finish(last, last % NBUF)
        out_cp[last % NBUF].wait()
        out_cp[(last - 1) % NBUF].wait()

    return sc_kernel


def kernel(feat_item, feat_category, feat_brand, positions,
           table_item, table_category, table_brand, table_position):
    batch, pos_len = feat_item.shape

    idx_a = feat_item.T.reshape(-1)
    idx_b = feat_category.T.reshape(-1)
    idx_c = feat_brand.T.reshape(-1)
    pos_rows = jnp.take(table_position, positions, axis=0)
    blk = batch // 16
    pos_rep = jnp.broadcast_to(pos_rows[:, None, :], (pos_len, blk, DIM))

    sc = _make_sc_kernel(batch, pos_len)
    out_t = sc(idx_a, idx_b, idx_c, table_item, table_category,
               table_brand, pos_rep)
    return jnp.transpose(out_t, (2, 0, 1))


# speedup vs baseline: 1.4110x; 1.3239x over previous
import functools

import jax
import jax.numpy as jnp
from jax import lax
from jax.experimental import pallas as pl
from jax.experimental.pallas import tpu as pltpu
from jax.experimental.pallas import tpu_sc as plsc

DIM = 64
CHUNK = 400
NBUF = 3


def _make_sc_kernel(n_rows: int, n_workers: int):
    rows_per_w = n_rows // n_workers
    n_chunks = rows_per_w // CHUNK
    mesh = plsc.VectorSubcoreMesh(core_axis_name="c", subcore_axis_name="s")

    @functools.partial(
        pl.kernel,
        mesh=mesh,
        compiler_params=pltpu.CompilerParams(use_tc_tiling_on_sc=False),
        out_type=jax.ShapeDtypeStruct((n_rows, DIM), jnp.float32),
        scratch_types=[
            [pltpu.VMEM((CHUNK,), jnp.int32)] * NBUF,
            [pltpu.VMEM((CHUNK,), jnp.int32)] * NBUF,
            [pltpu.VMEM((CHUNK,), jnp.int32)] * NBUF,
            [pltpu.VMEM((CHUNK, DIM), jnp.float32)] * NBUF,
            [pltpu.SemaphoreType.DMA] * NBUF,
            [pltpu.SemaphoreType.DMA] * NBUF,
        ],
    )
    def sc_kernel(idx_a_hbm, idx_b_hbm, idx_c_hbm, tab_a_hbm, tab_b_hbm,
                  tab_c_hbm, pos_hbm, out_hbm,
                  idx_a, idx_b, idx_c, acc, sem_g, sem_o):
        n_cores = 2
        wid = lax.axis_index("s") * n_cores + lax.axis_index("c")
        w_base = wid * rows_per_w

        gathers = [None] * NBUF
        out_cp = [None] * NBUF

        def chunk_slice(c):
            return pl.ds(w_base + c * CHUNK, CHUNK)

        for c in range(n_chunks):
            p = c % NBUF
            if out_cp[p] is not None:
                out_cp[p].wait()
            sl = chunk_slice(c)
            pltpu.sync_copy(idx_a_hbm.at[sl], idx_a[p])
            pltpu.sync_copy(idx_b_hbm.at[sl], idx_b[p])
            pltpu.sync_copy(idx_c_hbm.at[sl], idx_c[p])
            pltpu.sync_copy(pos_hbm, acc[p])
            gathers[p] = [
                pltpu.async_copy(tab_a_hbm.at[idx_a[p]], acc[p], sem_g[p], add=True),
                pltpu.async_copy(tab_b_hbm.at[idx_b[p]], acc[p], sem_g[p], add=True),
                pltpu.async_copy(tab_c_hbm.at[idx_c[p]], acc[p], sem_g[p], add=True),
            ]
            if c >= 1:
                q = (c - 1) % NBUF
                for d in gathers[q]:
                    d.wait()
                out_cp[q] = pltpu.async_copy(acc[q], out_hbm.at[chunk_slice(c - 1)], sem_o[q])
        last = (n_chunks - 1) % NBUF
        for d in gathers[last]:
            d.wait()
        pltpu.async_copy(acc[last], out_hbm.at[chunk_slice(n_chunks - 1)], sem_o[last]).wait()
        for c in range(max(n_chunks - NBUF, 0), n_chunks - 1):
            out_cp[c % NBUF].wait()

    return sc_kernel


def kernel(feat_item, feat_category, feat_brand, positions,
           table_item, table_category, table_brand, table_position):
    batch, pos_len = feat_item.shape
    n_rows = batch * pos_len

    idx_a = feat_item.reshape(n_rows)
    idx_b = feat_category.reshape(n_rows)
    idx_c = feat_brand.reshape(n_rows)

    pos_rows = jnp.take(table_position, positions, axis=0)
    pos_block = jnp.tile(pos_rows, (CHUNK // pos_len, 1))

    sc = _make_sc_kernel(n_rows, 32)
    out = sc(idx_a, idx_b, idx_c, table_item, table_category,
             table_brand, pos_block)
    return out.reshape(batch, pos_len, DIM)


# speedup vs baseline: 1.5886x; 1.1259x over previous
import functools

import jax
import jax.numpy as jnp
from jax import lax
from jax.experimental import pallas as pl
from jax.experimental.pallas import tpu as pltpu
from jax.experimental.pallas import tpu_sc as plsc

DIM = 64
CHUNK = 800
NBUF = 2


def _make_sc_kernel(n_rows: int, n_workers: int):
    rows_per_w = n_rows // n_workers
    n_chunks = rows_per_w // CHUNK
    mesh = plsc.VectorSubcoreMesh(core_axis_name="c", subcore_axis_name="s")

    @functools.partial(
        pl.kernel,
        mesh=mesh,
        compiler_params=pltpu.CompilerParams(use_tc_tiling_on_sc=False),
        out_type=jax.ShapeDtypeStruct((n_rows, DIM), jnp.float32),
        scratch_types=[
            [pltpu.VMEM((CHUNK,), jnp.int32)] * NBUF,
            [pltpu.VMEM((CHUNK,), jnp.int32)] * NBUF,
            [pltpu.VMEM((CHUNK,), jnp.int32)] * NBUF,
            [pltpu.VMEM((CHUNK, DIM), jnp.float32)] * NBUF,
            [pltpu.SemaphoreType.DMA] * NBUF,
            [pltpu.SemaphoreType.DMA] * NBUF,
        ],
    )
    def sc_kernel(idx_a_hbm, idx_b_hbm, idx_c_hbm, tab_a_hbm, tab_b_hbm,
                  tab_c_hbm, pos_hbm, out_hbm,
                  idx_a, idx_b, idx_c, acc, sem_g, sem_o):
        n_cores = 2
        wid = lax.axis_index("s") * n_cores + lax.axis_index("c")
        w_base = wid * rows_per_w

        gathers = [None] * NBUF
        out_cp = [None] * NBUF

        def chunk_slice(c):
            return pl.ds(w_base + c * CHUNK, CHUNK)

        for c in range(n_chunks):
            p = c % NBUF
            if out_cp[p] is not None:
                out_cp[p].wait()
            sl = chunk_slice(c)
            pltpu.sync_copy(idx_a_hbm.at[sl], idx_a[p])
            pltpu.sync_copy(idx_b_hbm.at[sl], idx_b[p])
            pltpu.sync_copy(idx_c_hbm.at[sl], idx_c[p])
            pltpu.sync_copy(pos_hbm, acc[p])
            gathers[p] = [
                pltpu.async_copy(tab_a_hbm.at[idx_a[p]], acc[p], sem_g[p], add=True),
                pltpu.async_copy(tab_b_hbm.at[idx_b[p]], acc[p], sem_g[p], add=True),
                pltpu.async_copy(tab_c_hbm.at[idx_c[p]], acc[p], sem_g[p], add=True),
            ]
            if c >= 1:
                q = (c - 1) % NBUF
                for d in gathers[q]:
                    d.wait()
                out_cp[q] = pltpu.async_copy(acc[q], out_hbm.at[chunk_slice(c - 1)], sem_o[q])
        last = (n_chunks - 1) % NBUF
        for d in gathers[last]:
            d.wait()
        pltpu.async_copy(acc[last], out_hbm.at[chunk_slice(n_chunks - 1)], sem_o[last]).wait()
        for c in range(max(n_chunks - NBUF, 0), n_chunks - 1):
            out_cp[c % NBUF].wait()

    return sc_kernel


def kernel(feat_item, feat_category, feat_brand, positions,
           table_item, table_category, table_brand, table_position):
    batch, pos_len = feat_item.shape
    n_rows = batch * pos_len

    idx_a = feat_item.reshape(n_rows)
    idx_b = feat_category.reshape(n_rows)
    idx_c = feat_brand.reshape(n_rows)

    pos_rows = jnp.take(table_position, positions, axis=0)
    pos_block = jnp.tile(pos_rows, (CHUNK // pos_len, 1))

    sc = _make_sc_kernel(n_rows, 32)
    out = sc(idx_a, idx_b, idx_c, table_item, table_category,
             table_brand, pos_block)
    return out.reshape(batch, pos_len, DIM)
